# baseline (speedup 1.0000x reference)
import functools

import jax
import jax.numpy as jnp
from jax import lax
from jax.experimental import pallas as pl
from jax.experimental.pallas import tpu as pltpu
from jax.experimental.pallas import tpu_sc as plsc

_NUM_GROUPS = 2
_D = 16
_NC, _NS = 2, 16
_NW = _NC * _NS
_PACK = 4
_GSUB = 104
_BCH = 16


@functools.partial(jax.jit, static_argnames=("batch", "fields"))
def _fm_sparsecore(idx_flat, vals_flat, emb4, batch, fields):
    b_per_w = batch // _NW
    n_per_w = b_per_w * fields
    chunk_n = _BCH * fields
    n_chunks = b_per_w // _BCH
    subs = chunk_n // _GSUB

    mesh = plsc.VectorSubcoreMesh(core_axis_name="c", subcore_axis_name="s",
                                  num_cores=_NC, num_subcores=_NS)

    @functools.partial(
        pl.kernel,
        out_type=jax.ShapeDtypeStruct((batch, _D), jnp.float32),
        mesh=mesh,
        compiler_params=pltpu.CompilerParams(use_tc_tiling_on_sc=False),
        scratch_types=[
            pltpu.VMEM((n_per_w,), jnp.int32),
            pltpu.VMEM((n_per_w,), jnp.int32),
            pltpu.VMEM((n_per_w + _D,), jnp.float32),
            pltpu.VMEM((2, chunk_n, 128), jnp.float32),
            pltpu.VMEM((b_per_w, _D), jnp.float32),
            pltpu.SemaphoreType.DMA,
        ],
    )
    def fm_kernel(idx_hbm, vals_hbm, emb_hbm, out_hbm,
                  idx_v, idx4_v, vals_v, rows_v, out_v, sem):
        wid = lax.axis_index("s") * _NC + lax.axis_index("c")
        pltpu.sync_copy(idx_hbm.at[pl.ds(wid * n_per_w, n_per_w)], idx_v)
        pltpu.sync_copy(vals_hbm.at[pl.ds(wid * n_per_w, n_per_w)],
                        vals_v.at[pl.ds(0, n_per_w)])

        def shift_body(i, carry):
            idx4_v[pl.ds(i * _D, _D)] = jax.lax.shift_right_logical(
                idx_v[pl.ds(i * _D, _D)], 2)
            return carry

        lax.fori_loop(0, n_per_w // _D, shift_body, 0)

        def fire(c):
            base = c * chunk_n
            buf = c % 2
            return [
                pltpu.async_copy(
                    emb_hbm.at[idx4_v.at[pl.ds(base + j * _GSUB, _GSUB)]],
                    rows_v.at[buf, pl.ds(j * _GSUB, _GSUB)], sem)
                for j in range(subs)
            ]

        def compute(c):
            buf = c % 2

            def body(b, carry):
                bb = c * _BCH + b
                base = bb * fields
                vv = [vals_v[pl.ds(base + k * _D, _D)]
                      for k in range((fields + _D - 1) // _D)]
                iv = [idx_v[pl.ds(base + k * _D, _D)]
                      for k in range((fields + _D - 1) // _D)]
                s = jnp.zeros((_D,), jnp.float32)
                q = jnp.zeros((_D,), jnp.float32)
                loc = b * fields
                for f in range(fields):
                    r = iv[f // _D][f % _D]
                    off = (r & 3) * _D
                    t = rows_v[buf, loc + f, pl.ds(off, _D)] \
                        * vv[f // _D][f % _D]
                    s = s + t
                    q = q + t * t
                out_v[bb] = 0.5 * (s * s - q)
                return carry

            lax.fori_loop(0, _BCH, body, 0)

        pend = fire(0)
        for c in range(n_chunks):
            for d in pend:
                d.wait()
            if c + 1 < n_chunks:
                pend = fire(c + 1)
            compute(c)

        pltpu.sync_copy(out_v, out_hbm.at[pl.ds(wid * b_per_w, b_per_w)])

    return fm_kernel(idx_flat, vals_flat, emb4)


def _tp_body(a_ref, o_ref):
    b = a_ref[...].T
    b3 = b.reshape(b.shape[0] // _PACK, _PACK, _D)
    for k in range(_PACK):
        o_ref[:, _D * k:_D * (k + 1)] = b3[:, k, :]


@jax.jit
def _row_major_table4(emb_t):
    v = emb_t.shape[1]
    blk_r = 4096
    blk_c = blk_r * _PACK
    grid = (v // _PACK + blk_r - 1) // blk_r
    return pl.pallas_call(
        _tp_body,
        grid=(grid,),
        in_specs=[pl.BlockSpec((16, blk_c), lambda g: (0, g))],
        out_specs=pl.BlockSpec((blk_r, 128), lambda g: (g, 0)),
        out_shape=jax.ShapeDtypeStruct((v // _PACK, 128), jnp.float32),
    )(emb_t)


def _mlp_body(fm_ref, w1_ref, b1_ref, w2_ref, b2_ref, wp_ref, out_ref):
    h = jnp.dot(fm_ref[...], w1_ref[...], preferred_element_type=jnp.float32)
    h = jnp.maximum(h + b1_ref[...], 0.0)
    h = jnp.dot(h, w2_ref[...], preferred_element_type=jnp.float32)
    h = jnp.maximum(h + b2_ref[...], 0.0)
    out_ref[...] = jnp.dot(h, wp_ref[...],
                           preferred_element_type=jnp.float32)


@jax.jit
def _mlp_tensorcore(fm, w1t, b1, w2t, b2, wpt):
    batch = fm.shape[0]
    return pl.pallas_call(
        _mlp_body,
        out_shape=jax.ShapeDtypeStruct((batch, 1), jnp.float32),
    )(fm, w1t, b1, w2t, b2, wpt)


def kernel(features, feature_values, emb, biases, bias_, W1, b1, W2, b2, Wp):
    batch, tot_fields = features.shape
    fields = tot_fields - _NUM_GROUPS
    idx_flat = features[:, :fields].reshape(-1)
    vals_flat = feature_values[:, :fields].reshape(-1)

    emb4 = _row_major_table4(emb.T)
    fm = _fm_sparsecore(idx_flat, vals_flat, emb4, batch, fields)
    out = _mlp_tensorcore(fm, W1.T, b1.reshape(1, -1), W2.T,
                          b2.reshape(1, -1), Wp.T)
    return (out + bias_).reshape(-1)


# speedup vs baseline: 1.3401x; 1.3401x over previous
import functools

import jax
import jax.numpy as jnp
from jax import lax
from jax.experimental import pallas as pl
from jax.experimental.pallas import tpu as pltpu
from jax.experimental.pallas import tpu_sc as plsc

_NUM_GROUPS = 2
_D = 16
_NC, _NS = 2, 16
_NW = _NC * _NS
_CHUNK = 128


@functools.partial(jax.jit, static_argnames=("batch", "fields"))
def _fm_sparsecore(idx_flat, vals_flat, emb, batch, fields):
    b_per_w = batch // _NW
    n_per_w = b_per_w * fields
    chunks_per_w = n_per_w // _CHUNK

    mesh = plsc.VectorSubcoreMesh(core_axis_name="c", subcore_axis_name="s",
                                  num_cores=_NC, num_subcores=_NS)

    @functools.partial(
        pl.kernel,
        out_type=jax.ShapeDtypeStruct((batch, _D), jnp.float32),
        mesh=mesh,
        compiler_params=pltpu.CompilerParams(use_tc_tiling_on_sc=False),
        scratch_types=[
            pltpu.VMEM((n_per_w,), jnp.int32),
            pltpu.VMEM((n_per_w + _D,), jnp.float32),
            pltpu.VMEM((n_per_w, _D), jnp.float32),
            pltpu.VMEM((b_per_w, _D), jnp.float32),
            pltpu.SemaphoreType.DMA,
        ],
    )
    def fm_kernel(idx_hbm, vals_hbm, emb_hbm, out_hbm,
                  idx_v, vals_v, rows_v, out_v, sem):
        wid = lax.axis_index("s") * _NC + lax.axis_index("c")
        pltpu.sync_copy(idx_hbm.at[pl.ds(wid * n_per_w, n_per_w)], idx_v)
        pltpu.sync_copy(vals_hbm.at[pl.ds(wid * n_per_w, n_per_w)],
                        vals_v.at[pl.ds(0, n_per_w)])
        copies = [
            pltpu.async_copy(emb_hbm.at[idx_v.at[pl.ds(j * _CHUNK, _CHUNK)]],
                             rows_v.at[pl.ds(j * _CHUNK, _CHUNK)], sem)
            for j in range(chunks_per_w)
        ]
        for c in copies:
            c.wait()

        def body(b, carry):
            s = jnp.zeros((_D,), jnp.float32)
            q = jnp.zeros((_D,), jnp.float32)
            base = b * fields
            vv = [vals_v[pl.ds(base + k * _D, _D)]
                  for k in range((fields + _D - 1) // _D)]
            for f in range(fields):
                t = rows_v[base + f] * vv[f // _D][f % _D]
                s = s + t
                q = q + t * t
            out_v[b] = 0.5 * (s * s - q)
            return carry

        lax.fori_loop(0, b_per_w, body, 0)
        pltpu.sync_copy(out_v, out_hbm.at[pl.ds(wid * b_per_w, b_per_w)])

    return fm_kernel(idx_flat, vals_flat, emb)


def _tp_body(a_ref, o_ref):
    b = a_ref[...].T
    b3 = b.reshape(b.shape[0] // 8, 8, 16)
    for k in range(8):
        o_ref[:, 16 * k:16 * (k + 1)] = b3[:, k, :]


@jax.jit
def _row_major_table(emb_t):
    v = emb_t.shape[1]
    blk_r = 2048
    blk_c = blk_r * 8
    grid = (v // 8 + blk_r - 1) // blk_r
    return pl.pallas_call(
        _tp_body,
        grid=(grid,),
        in_specs=[pl.BlockSpec((16, blk_c), lambda g: (0, g))],
        out_specs=pl.BlockSpec((blk_r, 128), lambda g: (g, 0)),
        out_shape=jax.ShapeDtypeStruct((v // 8, 128), jnp.float32),
    )(emb_t)


def _mlp_body(fm_ref, w1_ref, b1_ref, w2_ref, b2_ref, wp_ref, out_ref):
    h = jnp.dot(fm_ref[...], w1_ref[...], preferred_element_type=jnp.float32)
    h = jnp.maximum(h + b1_ref[...], 0.0)
    h = jnp.dot(h, w2_ref[...], preferred_element_type=jnp.float32)
    h = jnp.maximum(h + b2_ref[...], 0.0)
    out_ref[...] = jnp.dot(h, wp_ref[...],
                           preferred_element_type=jnp.float32)


@jax.jit
def _mlp_tensorcore(fm, w1t, b1, w2t, b2, wpt):
    batch = fm.shape[0]
    return pl.pallas_call(
        _mlp_body,
        out_shape=jax.ShapeDtypeStruct((batch, 1), jnp.float32),
    )(fm, w1t, b1, w2t, b2, wpt)


def kernel(features, feature_values, emb, biases, bias_, W1, b1, W2, b2, Wp):
    batch, tot_fields = features.shape
    fields = tot_fields - _NUM_GROUPS
    idx_flat = features[:, :fields].reshape(-1)
    vals_flat = feature_values[:, :fields].reshape(-1)

    emb_rm = _row_major_table(emb.T).reshape(emb.shape)
    fm = _fm_sparsecore(idx_flat, vals_flat, emb_rm, batch, fields)
    out = _mlp_tensorcore(fm, W1.T, b1.reshape(1, -1), W2.T,
                          b2.reshape(1, -1), Wp.T)
    return (out + bias_).reshape(-1)
